# Initial kernel scaffold; baseline (speedup 1.0000x reference)
import jax
import jax.numpy as jnp
from jax.experimental import pallas as pl


def kernel(inputs, embed, seq_len, aux_loss, is_training, Wr0, W1_0, b1_0, V0, a0, c0, Wp0, bp0, Wr1, W1_1, b1_1, V1, a1, c1, Wp1, bp1, Wq, bq, Wk, bk, Wv, bv, Wo, bo, mem_k, mem_v, ln_g, ln_b):
    raise NotImplementedError("write your pallas kernel here")


# speedup vs baseline: 2.4956x; 2.4956x over previous
import functools

import jax
import jax.numpy as jnp
from jax import lax
from jax.experimental import pallas as pl
from jax.experimental.pallas import tpu as pltpu
from jax.experimental.pallas import tpu_sc as plsc

B = 2; T = 2048; IN = 512; ED = 256; HID = 1024; MD = 512; H = 8; DH = 64
NMEM = 64; E = 8; LB = 5; LA = 2; SL = 2; SR = 1
N = B * T
C = N // E
EC = E * C
TB = 512
NBLK = N // TB
NW = 32
TOKW = N // NW


def _route_body(x_ref, e_ref, wrx_ref, wre_ref, sl_ref,
                dscat_ref, dgat_ref, gate_ref, counts_ref):
    i = pl.program_id(0)

    @pl.when(i == 0)
    def _():
        counts_ref[...] = jnp.zeros_like(counts_ref)

    logits = (jnp.dot(x_ref[...], wrx_ref[...], preferred_element_type=jnp.float32)
              + jnp.dot(e_ref[...], wre_ref[...], preferred_element_type=jnp.float32))
    rowmax = jnp.max(logits, axis=-1, keepdims=True)
    iota_e = lax.broadcasted_iota(jnp.int32, (TB, E), 1)
    idx = jnp.min(jnp.where(logits == rowmax, iota_e, E), axis=-1, keepdims=True)
    gate = 1.0 / jnp.sum(jnp.exp(logits - rowmax), axis=-1, keepdims=True)

    oh = (iota_e == idx).astype(jnp.float32)
    same = lax.dot_general(oh, oh, (((1,), (1,)), ((), ())),
                           preferred_element_type=jnp.float32)
    lt = (lax.broadcasted_iota(jnp.int32, (TB, TB), 1)
          < lax.broadcasted_iota(jnp.int32, (TB, TB), 0)).astype(jnp.float32)
    rank_local = jnp.sum(same * lt, axis=-1, keepdims=True)

    cnt_prev = jnp.sum(oh * counts_ref[...], axis=-1, keepdims=True)
    rank = (rank_local + cnt_prev).astype(jnp.int32)
    counts_ref[...] = counts_ref[...] + jnp.sum(oh, axis=0, keepdims=True)

    keep = rank < C
    dest = idx * C + rank
    b_idx = i // (T // TB)
    t_in_seq = (i % (T // TB)) * TB + lax.broadcasted_iota(jnp.int32, (TB, 1), 0)
    m = t_in_seq < sl_ref[b_idx]

    dscat_ref[...] = jnp.where(keep, dest, EC)
    dgat_ref[...] = jnp.where(keep, dest, idx * C + (C - 1))
    gate_ref[...] = jnp.where(keep & m, gate, 0.0)


def _route(xf, ef, wrx, wre, seq_len):
    din = xf.shape[1]
    return pl.pallas_call(
        _route_body,
        grid=(NBLK,),
        in_specs=[
            pl.BlockSpec((TB, din), lambda i: (i, 0)),
            pl.BlockSpec((TB, ED), lambda i: (i, 0)),
            pl.BlockSpec((din, E), lambda i: (0, 0)),
            pl.BlockSpec((ED, E), lambda i: (0, 0)),
            pl.BlockSpec(memory_space=pltpu.SMEM),
        ],
        out_specs=[
            pl.BlockSpec((TB, 1), lambda i: (i, 0)),
            pl.BlockSpec((TB, 1), lambda i: (i, 0)),
            pl.BlockSpec((TB, 1), lambda i: (i, 0)),
        ],
        out_shape=[
            jax.ShapeDtypeStruct((N, 1), jnp.int32),
            jax.ShapeDtypeStruct((N, 1), jnp.int32),
            jax.ShapeDtypeStruct((N, 1), jnp.float32),
        ],
        scratch_shapes=[pltpu.VMEM((1, E), jnp.float32)],
    )(xf, ef, wrx, wre, seq_len)


def _sc_scatter_rows(x, idx, nrows):
    d = x.shape[1]
    mesh = plsc.VectorSubcoreMesh(core_axis_name="c", subcore_axis_name="s")

    @functools.partial(
        pl.kernel, mesh=mesh,
        out_type=jax.ShapeDtypeStruct((nrows, d), x.dtype),
        scratch_types=[
            pltpu.VMEM((TOKW,), jnp.int32),
            pltpu.VMEM((TOKW, d), x.dtype),
            pltpu.SemaphoreType.DMA,
        ],
    )
    def k(x_hbm, idx_hbm, out_hbm, idx_v, rows_v, sem):
        wid = lax.axis_index("s") * 2 + lax.axis_index("c")
        base = wid * TOKW
        pltpu.sync_copy(idx_hbm.at[pl.ds(base, TOKW)], idx_v)
        pltpu.sync_copy(x_hbm.at[pl.ds(base, TOKW)], rows_v)
        pltpu.async_copy(rows_v, out_hbm.at[idx_v], sem).wait()

    return k(x, idx)


def _sc_gather_rows(table, idx):
    d = table.shape[1]
    mesh = plsc.VectorSubcoreMesh(core_axis_name="c", subcore_axis_name="s")

    @functools.partial(
        pl.kernel, mesh=mesh,
        out_type=jax.ShapeDtypeStruct((N, d), table.dtype),
        scratch_types=[
            pltpu.VMEM((TOKW,), jnp.int32),
            pltpu.VMEM((TOKW, d), table.dtype),
            pltpu.SemaphoreType.DMA,
        ],
    )
    def k(table_hbm, idx_hbm, out_hbm, idx_v, rows_v, sem):
        wid = lax.axis_index("s") * 2 + lax.axis_index("c")
        base = wid * TOKW
        pltpu.sync_copy(idx_hbm.at[pl.ds(base, TOKW)], idx_v)
        pltpu.async_copy(table_hbm.at[idx_v], rows_v, sem).wait()
        pltpu.sync_copy(rows_v, out_hbm.at[pl.ds(base, TOKW)])

    return k(table, idx)


def _mlp_body(buf_ref, w1_ref, b1_ref, v_ref, out_ref):
    h = jnp.maximum(
        jnp.dot(buf_ref[...], w1_ref[0], preferred_element_type=jnp.float32)
        + b1_ref[0], 0.0)
    out_ref[...] = jnp.dot(h, v_ref[0], preferred_element_type=jnp.float32)


def _mlp(buf, w1, b1, v):
    din = buf.shape[1]
    return pl.pallas_call(
        _mlp_body,
        grid=(E,),
        in_specs=[
            pl.BlockSpec((C, din), lambda e: (e, 0)),
            pl.BlockSpec((1, din, HID), lambda e: (e, 0, 0)),
            pl.BlockSpec((1, 1, HID), lambda e: (e, 0, 0)),
            pl.BlockSpec((1, HID, ED), lambda e: (e, 0, 0)),
        ],
        out_specs=pl.BlockSpec((C, ED), lambda e: (e, 0)),
        out_shape=jax.ShapeDtypeStruct((EC, ED), jnp.float32),
    )(buf, w1, b1, v)


def _fsmn_body(has_skip, *refs):
    if has_skip:
        g_ref, gate_ref, at_ref, ct_ref, wp_ref, bp_ref, xs_ref, out_ref = refs
    else:
        g_ref, gate_ref, at_ref, ct_ref, wp_ref, bp_ref, out_ref = refs
    p = g_ref[0] * gate_ref[0]
    phat = p
    for i in range(1, LB + 1):
        off = i * SL
        shifted = jnp.concatenate(
            [jnp.zeros((off, ED), jnp.float32), p[:T - off]], axis=0)
        phat = phat + at_ref[i - 1][None, :] * shifted
    for j in range(1, LA + 1):
        off = j * SR
        shifted = jnp.concatenate(
            [p[off:], jnp.zeros((off, ED), jnp.float32)], axis=0)
        phat = phat + ct_ref[j - 1][None, :] * shifted
    y = jnp.dot(phat, wp_ref[...], preferred_element_type=jnp.float32) + bp_ref[...]
    if has_skip:
        y = y + xs_ref[0]
    out_ref[0] = y


def _fsmn(g, gate, at, ct, wp, bp, xskip):
    has_skip = xskip is not None
    in_specs = [
        pl.BlockSpec((1, T, ED), lambda b: (b, 0, 0)),
        pl.BlockSpec((1, T, 1), lambda b: (b, 0, 0)),
        pl.BlockSpec((LB, ED), lambda b: (0, 0)),
        pl.BlockSpec((LA, ED), lambda b: (0, 0)),
        pl.BlockSpec((ED, MD), lambda b: (0, 0)),
        pl.BlockSpec((1, MD), lambda b: (0, 0)),
    ]
    args = [g, gate, at, ct, wp, bp]
    if has_skip:
        in_specs.append(pl.BlockSpec((1, T, MD), lambda b: (b, 0, 0)))
        args.append(xskip)
    return pl.pallas_call(
        functools.partial(_fsmn_body, has_skip),
        grid=(B,),
        in_specs=in_specs,
        out_specs=pl.BlockSpec((1, T, MD), lambda b: (b, 0, 0)),
        out_shape=jax.ShapeDtypeStruct((B, T, MD), jnp.float32),
    )(*args)


def _qkv_body(x_ref, w_ref, b_ref, out_ref):
    out_ref[...] = (jnp.dot(x_ref[...], w_ref[...],
                            preferred_element_type=jnp.float32) + b_ref[...])


def _qkv(x2, wqkv, bqkv):
    return pl.pallas_call(
        _qkv_body,
        grid=(NBLK,),
        in_specs=[
            pl.BlockSpec((TB, MD), lambda i: (i, 0)),
            pl.BlockSpec((MD, 3 * MD), lambda i: (0, 0)),
            pl.BlockSpec((1, 3 * MD), lambda i: (0, 0)),
        ],
        out_specs=pl.BlockSpec((TB, 3 * MD), lambda i: (i, 0)),
        out_shape=jax.ShapeDtypeStruct((N, 3 * MD), jnp.float32),
    )(x2, wqkv, bqkv)


def _attn_body(q_ref, k_ref, v_ref, mk_ref, mv_ref, sl_ref, out_ref):
    b = pl.program_id(0)
    keymask = lax.broadcasted_iota(jnp.int32, (TB, T), 1) < sl_ref[b]
    bias = jnp.where(keymask, 0.0, -1e9)
    outs = []
    for s in range(2):
        q = q_ref[0][:, s * DH:(s + 1) * DH]
        k = k_ref[0][:, s * DH:(s + 1) * DH]
        v = v_ref[0][:, s * DH:(s + 1) * DH]
        mk = mk_ref[...][:, s * DH:(s + 1) * DH]
        mv = mv_ref[...][:, s * DH:(s + 1) * DH]
        s1 = lax.dot_general(q, k, (((1,), (1,)), ((), ())),
                             preferred_element_type=jnp.float32) * (1.0 / 8.0)
        s1 = s1 + bias
        s2 = lax.dot_general(q, mk, (((1,), (1,)), ((), ())),
                             preferred_element_type=jnp.float32) * (1.0 / 8.0)
        m = jnp.maximum(jnp.max(s1, axis=-1, keepdims=True),
                        jnp.max(s2, axis=-1, keepdims=True))
        e1 = jnp.exp(s1 - m)
        e2 = jnp.exp(s2 - m)
        den = (jnp.sum(e1, axis=-1, keepdims=True)
               + jnp.sum(e2, axis=-1, keepdims=True))
        outs.append((jnp.dot(e1, v, preferred_element_type=jnp.float32)
                     + jnp.dot(e2, mv, preferred_element_type=jnp.float32)) / den)
    out_ref[0] = jnp.concatenate(outs, axis=1)


def _attn(qkv3, mem_k, mem_v, seq_len):
    qkv3 = qkv3.reshape(B, T, 3 * MD)
    nq = T // TB
    h2blk = 2 * DH
    nh2 = H // 2
    return pl.pallas_call(
        _attn_body,
        grid=(B, nh2, nq),
        in_specs=[
            pl.BlockSpec((1, TB, h2blk), lambda b, h, qi: (b, qi, h)),
            pl.BlockSpec((1, T, h2blk), lambda b, h, qi: (b, 0, nh2 + h)),
            pl.BlockSpec((1, T, h2blk), lambda b, h, qi: (b, 0, 2 * nh2 + h)),
            pl.BlockSpec((NMEM, h2blk), lambda b, h, qi: (0, h)),
            pl.BlockSpec((NMEM, h2blk), lambda b, h, qi: (0, h)),
            pl.BlockSpec(memory_space=pltpu.SMEM),
        ],
        out_specs=pl.BlockSpec((1, TB, h2blk), lambda b, h, qi: (b, qi, h)),
        out_shape=jax.ShapeDtypeStruct((B, T, MD), jnp.float32),
    )(qkv3, qkv3, qkv3, mem_k, mem_v, seq_len)


def _out_ln_body(o_ref, x_ref, wo_ref, bo_ref, g_ref, b_ref, out_ref):
    y = (x_ref[...] + jnp.dot(o_ref[...], wo_ref[...],
                              preferred_element_type=jnp.float32) + bo_ref[...])
    mu = jnp.mean(y, axis=-1, keepdims=True)
    yc = y - mu
    var = jnp.mean(yc * yc, axis=-1, keepdims=True)
    out_ref[...] = yc * lax.rsqrt(var + 1e-5) * g_ref[...] + b_ref[...]


def _out_ln(o, x2, wo, bo, ln_g, ln_b):
    return pl.pallas_call(
        _out_ln_body,
        grid=(NBLK,),
        in_specs=[
            pl.BlockSpec((TB, MD), lambda i: (i, 0)),
            pl.BlockSpec((TB, MD), lambda i: (i, 0)),
            pl.BlockSpec((MD, MD), lambda i: (0, 0)),
            pl.BlockSpec((1, MD), lambda i: (0, 0)),
            pl.BlockSpec((1, MD), lambda i: (0, 0)),
            pl.BlockSpec((1, MD), lambda i: (0, 0)),
        ],
        out_specs=pl.BlockSpec((TB, MD), lambda i: (i, 0)),
        out_shape=jax.ShapeDtypeStruct((N, MD), jnp.float32),
    )(o, x2, wo, bo, ln_g, ln_b)


def _moe_layer(xf, ef, seq_len, wr, w1, b1, v, at, ct, wp, bp, xskip):
    din = xf.shape[1]
    dscat, dgat, gate = _route(xf, ef, wr[:din], wr[din:], seq_len)
    buf = _sc_scatter_rows(xf, dscat.reshape(N), EC + 1)
    pexp = _mlp(buf, w1, b1.reshape(E, 1, HID), v)
    g = _sc_gather_rows(pexp, dgat.reshape(N))
    return _fsmn(g.reshape(B, T, ED), gate.reshape(B, T, 1), at, ct, wp, bp, xskip)


def kernel(inputs, embed, seq_len, aux_loss, is_training, Wr0, W1_0, b1_0, V0, a0, c0, Wp0, bp0, Wr1, W1_1, b1_1, V1, a1, c1, Wp1, bp1, Wq, bq, Wk, bk, Wv, bv, Wo, bo, mem_k, mem_v, ln_g, ln_b):
    xf = inputs.reshape(N, IN)
    ef = embed.reshape(N, ED)
    seq_len = seq_len.astype(jnp.int32)

    y0 = _moe_layer(xf, ef, seq_len, Wr0, W1_0, b1_0, V0,
                    a0.T, c0.T, Wp0, bp0.reshape(1, MD), None)
    y1 = _moe_layer(y0.reshape(N, MD), ef, seq_len, Wr1, W1_1, b1_1, V1,
                    a1.T, c1.T, Wp1, bp1.reshape(1, MD), y0)

    x2 = y1.reshape(N, MD)
    wqkv = jnp.concatenate([Wq, Wk, Wv], axis=1)
    bqkv = jnp.concatenate([bq, bk, bv]).reshape(1, 3 * MD)
    qkv3 = _qkv(x2, wqkv, bqkv)
    o = _attn(qkv3, mem_k, mem_v, seq_len)
    out = _out_ln(o.reshape(N, MD), x2, Wo, bo.reshape(1, MD),
                  ln_g.reshape(1, MD), ln_b.reshape(1, MD))
    return out.reshape(B, T, MD)
